# baseline (speedup 1.0000x reference)
import functools

import jax
import jax.numpy as jnp
from jax import lax
from jax.experimental import pallas as pl
from jax.experimental.pallas import tpu as pltpu
from jax.experimental.pallas import tpu_sc as plsc

VOCAB = 1000000
DIM = 64
B = 4096
L = 200

NC = 2
NS = 16
NW = NC * NS
SEQ_PER_W = B // NW
CHUNK = 104
LPAD = 2 * CHUNK


def _pool_body(idx_hbm, table_hbm, out_hbm, idx_v, buf0, buf1, out_v,
               sem0, sem1):
    c = lax.axis_index("c")
    s = lax.axis_index("s")
    wid = s * NC + c

    pltpu.sync_copy(idx_hbm.at[wid], idx_v)

    def fire(seq, buf, sem):
        pltpu.async_copy(table_hbm.at[idx_v.at[seq, 0]],
                         buf.at[pl.ds(0, CHUNK)], sem)
        pltpu.async_copy(table_hbm.at[idx_v.at[seq, 1]],
                         buf.at[pl.ds(CHUNK, CHUNK)], sem)

    def drain(buf, sem):
        pltpu.make_async_copy(table_hbm.at[idx_v.at[0, 0]],
                              buf.at[pl.ds(0, CHUNK)], sem).wait()
        pltpu.make_async_copy(table_hbm.at[idx_v.at[0, 1]],
                              buf.at[pl.ds(CHUNK, CHUNK)], sem).wait()

    fire(0, buf0, sem0)

    bufs = ((buf0, sem0), (buf1, sem1))

    def outer(g, carry):
        for par in range(2):
            buf, sem = bufs[par]
            nbuf, nsem = bufs[1 - par]
            seq = 2 * g + par
            drain(buf, sem)

            @pl.when(seq + 1 < SEQ_PER_W)
            def _():
                fire(seq + 1, nbuf, nsem)

            def acc_step(r, acc):
                return tuple(acc[d] + buf[r, pl.ds(16 * d, 16)]
                             for d in range(4))

            zero = jnp.zeros((16,), jnp.float32)
            a = lax.fori_loop(0, L, acc_step, (zero, zero, zero, zero))
            for d in range(4):
                out_v[seq, pl.ds(16 * d, 16)] = a[d]
        return carry

    lax.fori_loop(0, SEQ_PER_W // 2, outer, 0)
    pltpu.sync_copy(out_v, out_hbm.at[pl.ds(wid * SEQ_PER_W, SEQ_PER_W)])


@functools.partial(jax.jit, static_argnames=())
def _sc_pool(idx_arr, emb_table):
    mesh = plsc.VectorSubcoreMesh(core_axis_name="c", subcore_axis_name="s")
    return pl.kernel(
        _pool_body,
        mesh=mesh,
        compiler_params=pltpu.CompilerParams(use_tc_tiling_on_sc=False),
        out_type=jax.ShapeDtypeStruct((B, DIM), jnp.float32),
        scratch_types=[
            pltpu.VMEM((SEQ_PER_W, 2, CHUNK), jnp.int32),
            pltpu.VMEM((LPAD, DIM), jnp.float32),
            pltpu.VMEM((LPAD, DIM), jnp.float32),
            pltpu.VMEM((SEQ_PER_W, DIM), jnp.float32),
            pltpu.SemaphoreType.DMA,
            pltpu.SemaphoreType.DMA,
        ],
    )(idx_arr, emb_table)


def _mm_body(x_ref, wt_ref, b_ref, o_ref):
    o_ref[...] = jnp.dot(x_ref[...], wt_ref[...],
                         preferred_element_type=jnp.float32) + b_ref[...]


def _tc_linear(pooled, wt, b2d):
    return pl.pallas_call(
        _mm_body,
        out_shape=jax.ShapeDtypeStruct((B, DIM), jnp.float32),
    )(pooled, wt, b2d)


def kernel(tokens, emb_table, W, b):
    pad = jnp.zeros((B, LPAD - L), jnp.int32)
    idx_arr = jnp.concatenate([tokens, pad], axis=1).reshape(
        NW, SEQ_PER_W, 2, CHUNK)
    pooled = _sc_pool(idx_arr, emb_table)
    wt = (W.T * (1.0 / L)).astype(jnp.float32)
    return _tc_linear(pooled, wt, b.reshape(1, DIM))
